# baseline (speedup 1.0000x reference)
import functools

import jax
import jax.numpy as jnp
from jax import lax
from jax.experimental import pallas as pl
from jax.experimental.pallas import tpu as pltpu
from jax.experimental.pallas import tpu_sc as plsc

_N = 8192
_D = 64
_TOPK = 16
_ROWS = 512
_LANES = 128
_LAYERS = _N // _LANES
_DEPTH = 8
_NEG = -2.0


def _ce(a, b):
    av, aj = a
    bv, bj = b
    c = av >= bv
    hi = (jnp.where(c, av, bv), jnp.where(c, aj, bj))
    lo = (jnp.where(c, bv, av), jnp.where(c, bj, aj))
    return hi, lo


_BATCHER8 = [
    (0, 1), (2, 3), (4, 5), (6, 7),
    (0, 2), (1, 3), (4, 6), (5, 7),
    (1, 2), (5, 6),
    (0, 4), (1, 5), (2, 6), (3, 7),
    (2, 4), (3, 5),
    (1, 2), (3, 4), (5, 6),
]


def _sort8(a):
    for i, j in _BATCHER8:
        a[i], a[j] = _ce(a[i], a[j])


def _bmerge(a, lo, n, desc):
    if n > 1:
        m = n // 2
        for i in range(lo, lo + m):
            if desc:
                a[i], a[i + m] = _ce(a[i], a[i + m])
            else:
                a[i + m], a[i] = _ce(a[i + m], a[i])
        _bmerge(a, lo, m, desc)
        _bmerge(a, lo + m, m, desc)


def _merge_trunc(A, B):
    C = []
    for i in range(_DEPTH):
        av, aj = A[i]
        bv, bj = B[_DEPTH - 1 - i]
        c = av >= bv
        C.append((jnp.where(c, av, bv), jnp.where(c, aj, bj)))
    _bmerge(C, 0, _DEPTH, True)
    return C


def _norm_body(x_ref, xn_ref):
    x = x_ref[...]
    n2 = jnp.sum(x * x, axis=1, keepdims=True)
    xn_ref[...] = x / jnp.maximum(jnp.sqrt(n2), 1e-12)


def _topk_body(xr_ref, xa_ref, vals_ref, idx_ref):
    s = jax.lax.dot_general(
        xr_ref[...], xa_ref[...], (((1,), (1,)), ((), ())),
        preferred_element_type=jnp.float32,
    )

    pairs = [
        (s[:, _LANES * j:_LANES * (j + 1)],
         jnp.full((_ROWS, _LANES), j, jnp.int32))
        for j in range(_LAYERS)
    ]
    run = None
    for g in range(_LAYERS // _DEPTH):
        grp = pairs[_DEPTH * g:_DEPTH * (g + 1)]
        _sort8(grp)
        run = grp if run is None else _merge_trunc(run, grp)

    lane = jax.lax.broadcasted_iota(jnp.int32, (_ROWS, _LANES), 1)
    T = [(tv, tj * _LANES + lane) for (tv, tj) in run]
    big = jnp.int32(2**30)
    vals_l, idx_l = [], []
    for _ in range(_TOPK + 1):
        hv, hc = T[0]
        m = jnp.max(hv, axis=1, keepdims=True)
        eq = hv == m
        ix = jnp.min(jnp.where(eq, hc, big), axis=1, keepdims=True)
        vals_l.append(m)
        idx_l.append(ix)
        pop = hc == ix
        for d in range(_DEPTH - 1):
            T[d] = (
                jnp.where(pop, T[d + 1][0], T[d][0]),
                jnp.where(pop, T[d + 1][1], T[d][1]),
            )
        T[_DEPTH - 1] = (
            jnp.where(pop, _NEG, T[_DEPTH - 1][0]),
            T[_DEPTH - 1][1],
        )
    vals_ref[...] = jnp.concatenate(vals_l[1:], axis=1)
    idx_ref[...] = jnp.concatenate(idx_l[1:], axis=1)


def _knn_topk(x):
    xn = pl.pallas_call(
        _norm_body,
        out_shape=jax.ShapeDtypeStruct((_N, _D), jnp.float32),
    )(x)
    return pl.pallas_call(
        _topk_body,
        grid=(_N // _ROWS,),
        in_specs=[
            pl.BlockSpec((_ROWS, _D), lambda i: (i, 0)),
            pl.BlockSpec((_N, _D), lambda i: (0, 0)),
        ],
        out_specs=[
            pl.BlockSpec((_ROWS, _TOPK), lambda i: (i, 0)),
            pl.BlockSpec((_ROWS, _TOPK), lambda i: (i, 0)),
        ],
        out_shape=[
            jax.ShapeDtypeStruct((_N, _TOPK), jnp.float32),
            jax.ShapeDtypeStruct((_N, _TOPK), jnp.int32),
        ],
    )(xn, xn)


_E = 131072
_TOTAL = 4 * _E + _N


def _sc_assemble(ei0, ei1, nbr_flat, kres16):
    info = plsc.get_sparse_core_info()
    NC, NS = info.num_cores, info.num_subcores
    NW = NC * NS
    CH = _E // NW
    CHL = _N // NW
    mesh = plsc.VectorSubcoreMesh(core_axis_name="c", subcore_axis_name="s")

    @functools.partial(
        pl.kernel,
        mesh=mesh,
        out_type=[
            jax.ShapeDtypeStruct((_TOTAL,), jnp.int32),
            jax.ShapeDtypeStruct((_TOTAL,), jnp.int32),
        ],
        scratch_types=[
            pltpu.VMEM((CH,), jnp.int32),
            pltpu.VMEM((CH,), jnp.int32),
            pltpu.VMEM((16,), jnp.int32),
            pltpu.VMEM((CHL,), jnp.int32),
        ],
    )
    def k(ei0_h, ei1_h, nbr_h, kres_h, out0_h, out1_h, buf, gen, kv, lpb):
        wid = lax.axis_index("s") * NC + lax.axis_index("c")
        base = wid * CH
        pltpu.sync_copy(kres_h, kv)
        kvv = kv[...]
        pltpu.sync_copy(ei0_h.at[pl.ds(base, CH)], buf)
        pltpu.sync_copy(buf, out0_h.at[pl.ds(base, CH)])
        pltpu.sync_copy(buf, out1_h.at[pl.ds(2 * _E + base, CH)])
        pltpu.sync_copy(ei1_h.at[pl.ds(base, CH)], buf)
        pltpu.sync_copy(buf, out1_h.at[pl.ds(base, CH)])
        pltpu.sync_copy(buf, out0_h.at[pl.ds(2 * _E + base, CH)])
        pltpu.sync_copy(nbr_h.at[pl.ds(base, CH)], buf)
        pltpu.sync_copy(buf, out0_h.at[pl.ds(3 * _E + base, CH)])
        pltpu.sync_copy(buf, out1_h.at[pl.ds(_E + base, CH)])

        def gen_body(j, carry):
            vec = ((jnp.arange(16, dtype=jnp.int32) + (base + 16 * j)) >> 4) + kvv
            gen[pl.ds(16 * j, 16)] = vec
            return carry

        lax.fori_loop(0, CH // 16, gen_body, 0)
        pltpu.sync_copy(gen, out0_h.at[pl.ds(_E + base, CH)])
        pltpu.sync_copy(gen, out1_h.at[pl.ds(3 * _E + base, CH)])

        lbase = wid * CHL

        def lp_body(j, carry):
            lpb[pl.ds(16 * j, 16)] = jnp.arange(16, dtype=jnp.int32) + (
                lbase + 16 * j
            )
            return carry

        lax.fori_loop(0, CHL // 16, lp_body, 0)
        pltpu.sync_copy(lpb, out0_h.at[pl.ds(4 * _E + lbase, CHL)])
        pltpu.sync_copy(lpb, out1_h.at[pl.ds(4 * _E + lbase, CHL)])

    r0, r1 = k(ei0, ei1, nbr_flat, kres16)
    return jnp.stack([r0, r1], axis=0)


def kernel(x, edge_index, k, feature_weight, geographic_weight, global_weight):
    num_nodes = x.shape[0]
    nbr_vals, nbr_idx = _knn_topk(x)
    adaptive_k = (
        min(_TOPK, num_nodes // 4) if num_nodes > 20 else min(_TOPK, num_nodes - 1)
    )
    k_residual = (jnp.asarray(k) - adaptive_k).astype(edge_index.dtype)
    kres16 = jnp.full((16,), k_residual, dtype=jnp.int32)
    final_edge_index = _sc_assemble(
        edge_index[0], edge_index[1], nbr_idx.reshape(-1), kres16
    )
    edge_weights = feature_weight * nbr_vals
    return final_edge_index, edge_weights


# speedup vs baseline: 1.0351x; 1.0351x over previous
import functools

import jax
import jax.numpy as jnp
from jax import lax
from jax.experimental import pallas as pl
from jax.experimental.pallas import tpu as pltpu
from jax.experimental.pallas import tpu_sc as plsc

_N = 8192
_D = 64
_TOPK = 16
_ROWS = 512
_LANES = 128
_LAYERS = _N // _LANES
_DEPTH = 8
_NEG = -2.0


def _ce(a, b):
    av, aj = a
    bv, bj = b
    c = av >= bv
    hi = (jnp.where(c, av, bv), jnp.where(c, aj, bj))
    lo = (jnp.where(c, bv, av), jnp.where(c, bj, aj))
    return hi, lo


_BATCHER8 = [
    (0, 1), (2, 3), (4, 5), (6, 7),
    (0, 2), (1, 3), (4, 6), (5, 7),
    (1, 2), (5, 6),
    (0, 4), (1, 5), (2, 6), (3, 7),
    (2, 4), (3, 5),
    (1, 2), (3, 4), (5, 6),
]


def _sort8(a):
    for i, j in _BATCHER8:
        a[i], a[j] = _ce(a[i], a[j])


def _bmerge(a, lo, n, desc):
    if n > 1:
        m = n // 2
        for i in range(lo, lo + m):
            if desc:
                a[i], a[i + m] = _ce(a[i], a[i + m])
            else:
                a[i + m], a[i] = _ce(a[i + m], a[i])
        _bmerge(a, lo, m, desc)
        _bmerge(a, lo + m, m, desc)


def _merge_trunc(A, B):
    C = []
    for i in range(_DEPTH):
        av, aj = A[i]
        bv, bj = B[_DEPTH - 1 - i]
        c = av >= bv
        C.append((jnp.where(c, av, bv), jnp.where(c, aj, bj)))
    _bmerge(C, 0, _DEPTH, True)
    return C


def _norm_body(x_ref, xn_ref):
    x = x_ref[...]
    n2 = jnp.sum(x * x, axis=1, keepdims=True)
    xn_ref[...] = x / jnp.maximum(jnp.sqrt(n2), 1e-12)


def _topk_body(xr_ref, xa_ref, vals_ref, idx_ref):
    s = jax.lax.dot_general(
        xr_ref[...], xa_ref[...], (((1,), (1,)), ((), ())),
        preferred_element_type=jnp.float32,
    )

    pairs = [
        (s[:, _LANES * j:_LANES * (j + 1)],
         jnp.full((_ROWS, _LANES), j, jnp.int32))
        for j in range(_LAYERS)
    ]
    groups = []
    for g in range(_LAYERS // _DEPTH):
        grp = pairs[_DEPTH * g:_DEPTH * (g + 1)]
        _sort8(grp)
        groups.append(grp)
    while len(groups) > 1:
        groups = [
            _merge_trunc(groups[2 * m], groups[2 * m + 1])
            for m in range(len(groups) // 2)
        ]

    lane = jax.lax.broadcasted_iota(jnp.int32, (_ROWS, _LANES), 1)
    T = [(tv, tj * _LANES + lane) for (tv, tj) in groups[0]]
    big = jnp.int32(2**30)
    vals_l, idx_l = [], []
    for _ in range(_TOPK + 1):
        hv, hc = T[0]
        m = jnp.max(hv, axis=1, keepdims=True)
        eq = hv == m
        ix = jnp.min(jnp.where(eq, hc, big), axis=1, keepdims=True)
        vals_l.append(m)
        idx_l.append(ix)
        pop = hc == ix
        for d in range(_DEPTH - 1):
            T[d] = (
                jnp.where(pop, T[d + 1][0], T[d][0]),
                jnp.where(pop, T[d + 1][1], T[d][1]),
            )
        T[_DEPTH - 1] = (
            jnp.where(pop, _NEG, T[_DEPTH - 1][0]),
            T[_DEPTH - 1][1],
        )
    vals_ref[...] = jnp.concatenate(vals_l[1:], axis=1)
    idx_ref[...] = jnp.concatenate(idx_l[1:], axis=1)


def _knn_topk(x):
    xn = pl.pallas_call(
        _norm_body,
        out_shape=jax.ShapeDtypeStruct((_N, _D), jnp.float32),
    )(x)
    return pl.pallas_call(
        _topk_body,
        grid=(_N // _ROWS,),
        in_specs=[
            pl.BlockSpec((_ROWS, _D), lambda i: (i, 0)),
            pl.BlockSpec((_N, _D), lambda i: (0, 0)),
        ],
        out_specs=[
            pl.BlockSpec((_ROWS, _TOPK), lambda i: (i, 0)),
            pl.BlockSpec((_ROWS, _TOPK), lambda i: (i, 0)),
        ],
        out_shape=[
            jax.ShapeDtypeStruct((_N, _TOPK), jnp.float32),
            jax.ShapeDtypeStruct((_N, _TOPK), jnp.int32),
        ],
    )(xn, xn)


_E = 131072
_TOTAL = 4 * _E + _N


def _sc_assemble(ei0, ei1, nbr_flat, kres16):
    info = plsc.get_sparse_core_info()
    NC, NS = info.num_cores, info.num_subcores
    NW = NC * NS
    CH = _E // NW
    CHL = _N // NW
    mesh = plsc.VectorSubcoreMesh(core_axis_name="c", subcore_axis_name="s")

    @functools.partial(
        pl.kernel,
        mesh=mesh,
        out_type=[
            jax.ShapeDtypeStruct((_TOTAL,), jnp.int32),
            jax.ShapeDtypeStruct((_TOTAL,), jnp.int32),
        ],
        scratch_types=[
            pltpu.VMEM((CH,), jnp.int32),
            pltpu.VMEM((CH,), jnp.int32),
            pltpu.VMEM((16,), jnp.int32),
            pltpu.VMEM((CHL,), jnp.int32),
        ],
    )
    def k(ei0_h, ei1_h, nbr_h, kres_h, out0_h, out1_h, buf, gen, kv, lpb):
        wid = lax.axis_index("s") * NC + lax.axis_index("c")
        base = wid * CH
        pltpu.sync_copy(kres_h, kv)
        kvv = kv[...]
        pltpu.sync_copy(ei0_h.at[pl.ds(base, CH)], buf)
        pltpu.sync_copy(buf, out0_h.at[pl.ds(base, CH)])
        pltpu.sync_copy(buf, out1_h.at[pl.ds(2 * _E + base, CH)])
        pltpu.sync_copy(ei1_h.at[pl.ds(base, CH)], buf)
        pltpu.sync_copy(buf, out1_h.at[pl.ds(base, CH)])
        pltpu.sync_copy(buf, out0_h.at[pl.ds(2 * _E + base, CH)])
        pltpu.sync_copy(nbr_h.at[pl.ds(base, CH)], buf)
        pltpu.sync_copy(buf, out0_h.at[pl.ds(3 * _E + base, CH)])
        pltpu.sync_copy(buf, out1_h.at[pl.ds(_E + base, CH)])

        def gen_body(j, carry):
            vec = ((jnp.arange(16, dtype=jnp.int32) + (base + 16 * j)) >> 4) + kvv
            gen[pl.ds(16 * j, 16)] = vec
            return carry

        lax.fori_loop(0, CH // 16, gen_body, 0)
        pltpu.sync_copy(gen, out0_h.at[pl.ds(_E + base, CH)])
        pltpu.sync_copy(gen, out1_h.at[pl.ds(3 * _E + base, CH)])

        lbase = wid * CHL

        def lp_body(j, carry):
            lpb[pl.ds(16 * j, 16)] = jnp.arange(16, dtype=jnp.int32) + (
                lbase + 16 * j
            )
            return carry

        lax.fori_loop(0, CHL // 16, lp_body, 0)
        pltpu.sync_copy(lpb, out0_h.at[pl.ds(4 * _E + lbase, CHL)])
        pltpu.sync_copy(lpb, out1_h.at[pl.ds(4 * _E + lbase, CHL)])

    r0, r1 = k(ei0, ei1, nbr_flat, kres16)
    return jnp.stack([r0, r1], axis=0)


def kernel(x, edge_index, k, feature_weight, geographic_weight, global_weight):
    num_nodes = x.shape[0]
    nbr_vals, nbr_idx = _knn_topk(x)
    adaptive_k = (
        min(_TOPK, num_nodes // 4) if num_nodes > 20 else min(_TOPK, num_nodes - 1)
    )
    k_residual = (jnp.asarray(k) - adaptive_k).astype(edge_index.dtype)
    kres16 = jnp.full((16,), k_residual, dtype=jnp.int32)
    final_edge_index = _sc_assemble(
        edge_index[0], edge_index[1], nbr_idx.reshape(-1), kres16
    )
    edge_weights = feature_weight * nbr_vals
    return final_edge_index, edge_weights
